# baseline (speedup 1.0000x reference)
import functools

import jax
import jax.numpy as jnp
from jax import lax
from jax.experimental import pallas as pl
from jax.experimental.pallas import tpu as pltpu, tpu_sc as plsc

B, C, H_p, W_p = 2, 192, 24, 24
H_img, W_img = 384, 384
NUM_SEG = 128
N_PATCH = H_p * W_p
N_PIX = H_img * W_img

NW = 32
N_ITEM = B * 48
ITEM_PIX = 3072
ITEMS_PER_W = N_ITEM // NW
CB = 16


def _means_body(feat_ref, seg_ref, out_ref):
    seg = jnp.clip(seg_ref[0], 0, NUM_SEG - 1)
    iota = jax.lax.broadcasted_iota(jnp.int32, (NUM_SEG, N_PATCH), 0)
    onehot = (iota == seg).astype(jnp.float32)
    sums = jax.lax.dot_general(
        onehot, feat_ref[0],
        dimension_numbers=(((1,), (1,)), ((), ())),
        preferred_element_type=jnp.float32,
        precision=jax.lax.Precision.HIGHEST)
    counts = jnp.sum(onehot, axis=1)
    out_ref[0] = sums / jnp.maximum(counts, 1.0)[:, None]


_sc_mesh = plsc.VectorSubcoreMesh(core_axis_name="c", subcore_axis_name="s")


@functools.partial(
    pl.kernel,
    mesh=_sc_mesh,
    compiler_params=pltpu.CompilerParams(needs_layout_passes=False),
    out_type=jax.ShapeDtypeStruct((B, C, 48, ITEM_PIX), jnp.float32),
    scratch_types=[
        pltpu.VMEM((B * NUM_SEG * C,), jnp.float32),
        pltpu.VMEM((ITEM_PIX,), jnp.int32),
        pltpu.VMEM((CB, ITEM_PIX), jnp.float32),
    ],
)
def _sc_paint(gid_hbm, means_hbm, out_hbm, means_v, gid_v, out_v):
    wid = lax.axis_index("s") * 2 + lax.axis_index("c")
    pltpu.sync_copy(means_hbm, means_v)

    def item_body(k, _):
        item = wid * ITEMS_PER_W + k
        b = item // 48
        ht = item % 48
        pltpu.sync_copy(gid_hbm.at[item], gid_v)

        def cblk_body(cb, _):
            def g_body(g, _):
                base16 = gid_v[pl.ds(g * 16, 16)] * C + cb * CB
                for ci in range(CB):
                    vals = plsc.load_gather(means_v, [base16 + ci])
                    out_v[ci, pl.ds(g * 16, 16)] = vals
                return 0

            lax.fori_loop(0, ITEM_PIX // 16, g_body, 0)
            pltpu.sync_copy(out_v, out_hbm.at[b, pl.ds(cb * CB, CB), ht])
            return 0

        lax.fori_loop(0, C // CB, cblk_body, 0)
        return 0

    lax.fori_loop(0, ITEMS_PER_W, item_body, 0)


@jax.jit
def kernel(F_semantic_patches, segmentation_mask):
    feat = F_semantic_patches.reshape(B, C, N_PATCH)
    seg_small = segmentation_mask[:, ::16, ::16].reshape(B, 1, N_PATCH)

    means = pl.pallas_call(
        _means_body,
        grid=(B,),
        in_specs=[
            pl.BlockSpec((1, C, N_PATCH), lambda b: (b, 0, 0)),
            pl.BlockSpec((1, 1, N_PATCH), lambda b: (b, 0, 0)),
        ],
        out_specs=pl.BlockSpec((1, NUM_SEG, C), lambda b: (b, 0, 0)),
        out_shape=jax.ShapeDtypeStruct((B, NUM_SEG, C), jnp.float32),
    )(feat, seg_small)
    means_flat = means.reshape(B * NUM_SEG * C)

    segc = jnp.clip(segmentation_mask, 0, NUM_SEG - 1)
    gid = segc + (jnp.arange(B, dtype=jnp.int32) * NUM_SEG)[:, None, None]
    gid6 = gid.reshape(B, 48, 8, 3, 128).transpose(0, 1, 3, 2, 4)
    gid_items = gid6.reshape(N_ITEM, ITEM_PIX)

    painted = _sc_paint(gid_items, means_flat)
    out = painted.reshape(B, C, 48, 3, 8, 128).transpose(0, 1, 2, 4, 3, 5)
    return out.reshape(B, C, H_img, W_img)


# speedup vs baseline: 1.0037x; 1.0037x over previous
import functools

import jax
import jax.numpy as jnp
from jax import lax
from jax.experimental import pallas as pl
from jax.experimental.pallas import tpu as pltpu, tpu_sc as plsc

B, C, H_p, W_p = 2, 192, 24, 24
H_img, W_img = 384, 384
NUM_SEG = 128
N_PATCH = H_p * W_p
N_PIX = H_img * W_img

NW = 32
N_ITEM = B * 48
ITEM_PIX = 3072
ITEMS_PER_W = N_ITEM // NW
CB = 16


def _means_body(feat_ref, seg_ref, out_ref):
    seg = jnp.clip(seg_ref[0], 0, NUM_SEG - 1)
    iota = jax.lax.broadcasted_iota(jnp.int32, (NUM_SEG, N_PATCH), 0)
    onehot = (iota == seg).astype(jnp.float32)
    sums = jax.lax.dot_general(
        onehot, feat_ref[0],
        dimension_numbers=(((1,), (1,)), ((), ())),
        preferred_element_type=jnp.float32,
        precision=jax.lax.Precision.HIGHEST)
    counts = jnp.sum(onehot, axis=1)
    out_ref[0] = sums / jnp.maximum(counts, 1.0)[:, None]


_sc_mesh = plsc.VectorSubcoreMesh(core_axis_name="c", subcore_axis_name="s")


@functools.partial(
    pl.kernel,
    mesh=_sc_mesh,
    compiler_params=pltpu.CompilerParams(needs_layout_passes=False),
    out_type=jax.ShapeDtypeStruct((B, C, 48, ITEM_PIX), jnp.float32),
    scratch_types=[
        pltpu.VMEM((B * NUM_SEG * C,), jnp.float32),
        pltpu.VMEM((ITEM_PIX,), jnp.int32),
        pltpu.VMEM((CB, ITEM_PIX), jnp.float32),
    ],
)
def _sc_paint(gid_hbm, means_hbm, out_hbm, means_v, gid_v, out_v):
    wid = lax.axis_index("s") * 2 + lax.axis_index("c")
    pltpu.sync_copy(means_hbm, means_v)

    def item_body(k, _):
        item = wid * ITEMS_PER_W + k
        b = item // 48
        ht = item % 48
        pltpu.sync_copy(gid_hbm.at[item], gid_v)

        def cblk_body(cb, _):
            def g_body(g, _):
                base16 = gid_v[pl.ds(g * 16, 16)] + cb * CB
                for ci in range(CB):
                    vals = plsc.load_gather(means_v, [base16 + ci])
                    out_v[ci, pl.ds(g * 16, 16)] = vals
                return 0

            lax.fori_loop(0, ITEM_PIX // 16, g_body, 0, unroll=4)
            pltpu.sync_copy(out_v, out_hbm.at[b, pl.ds(cb * CB, CB), ht])
            return 0

        lax.fori_loop(0, C // CB, cblk_body, 0)
        return 0

    lax.fori_loop(0, ITEMS_PER_W, item_body, 0)


@jax.jit
def kernel(F_semantic_patches, segmentation_mask):
    feat = F_semantic_patches.reshape(B, C, N_PATCH)
    seg_small = segmentation_mask[:, ::16, ::16].reshape(B, 1, N_PATCH)

    means = pl.pallas_call(
        _means_body,
        grid=(B,),
        in_specs=[
            pl.BlockSpec((1, C, N_PATCH), lambda b: (b, 0, 0)),
            pl.BlockSpec((1, 1, N_PATCH), lambda b: (b, 0, 0)),
        ],
        out_specs=pl.BlockSpec((1, NUM_SEG, C), lambda b: (b, 0, 0)),
        out_shape=jax.ShapeDtypeStruct((B, NUM_SEG, C), jnp.float32),
    )(feat, seg_small)
    means_flat = means.reshape(B * NUM_SEG * C)

    segc = jnp.clip(segmentation_mask, 0, NUM_SEG - 1)
    gid = segc * C + (jnp.arange(B, dtype=jnp.int32) * (NUM_SEG * C))[:, None, None]
    gid6 = gid.reshape(B, 48, 8, 3, 128).transpose(0, 1, 3, 2, 4)
    gid_items = gid6.reshape(N_ITEM, ITEM_PIX)

    painted = _sc_paint(gid_items, means_flat)
    out = painted.reshape(B, C, 48, 3, 8, 128).transpose(0, 1, 2, 4, 3, 5)
    return out.reshape(B, C, H_img, W_img)


# speedup vs baseline: 16.4661x; 16.4055x over previous
import functools

import jax
import jax.numpy as jnp
from jax.experimental import pallas as pl

B, C, H_p, W_p = 2, 192, 24, 24
H_img, W_img = 384, 384
NUM_SEG = 128
N_PATCH = H_p * W_p
N_PIX = H_img * W_img
PIX_BLK = 18432
ROWS_BLK = PIX_BLK // W_img
N_BLK = N_PIX // PIX_BLK


def _means_body(feat_ref, seg_ref, out_ref):
    seg = jnp.clip(seg_ref[0], 0, NUM_SEG - 1)
    iota = jax.lax.broadcasted_iota(jnp.int32, (NUM_SEG, N_PATCH), 0)
    onehot = (iota == seg).astype(jnp.float32)
    sums_t = jax.lax.dot_general(
        feat_ref[0], onehot,
        dimension_numbers=(((1,), (1,)), ((), ())),
        preferred_element_type=jnp.float32,
        precision=jax.lax.Precision.HIGHEST)
    counts = jnp.sum(onehot, axis=1)
    means = sums_t / jnp.maximum(counts, 1.0)[None, :]
    out_ref[0] = means.astype(jnp.bfloat16)


def _paint_body(seg_ref, means_ref, out_ref):
    seg = seg_ref[0]
    iota = jax.lax.broadcasted_iota(jnp.int16, (NUM_SEG, PIX_BLK), 0)
    onehot = (iota == seg).astype(jnp.bfloat16)
    out = jax.lax.dot_general(
        means_ref[0], onehot,
        dimension_numbers=(((1,), (0,)), ((), ())),
        preferred_element_type=jnp.float32)
    out_ref[0] = out.reshape(C, ROWS_BLK, W_img)


@jax.jit
def kernel(F_semantic_patches, segmentation_mask):
    feat = F_semantic_patches.reshape(B, C, N_PATCH)
    seg_small = segmentation_mask[:, ::16, ::16].reshape(B, 1, N_PATCH)

    means_t = pl.pallas_call(
        _means_body,
        grid=(B,),
        in_specs=[
            pl.BlockSpec((1, C, N_PATCH), lambda b: (b, 0, 0)),
            pl.BlockSpec((1, 1, N_PATCH), lambda b: (b, 0, 0)),
        ],
        out_specs=pl.BlockSpec((1, C, NUM_SEG), lambda b: (b, 0, 0)),
        out_shape=jax.ShapeDtypeStruct((B, C, NUM_SEG), jnp.bfloat16),
    )(feat, seg_small)

    seg_i16 = jnp.clip(segmentation_mask, 0, NUM_SEG - 1).astype(jnp.int16)
    seg_blk = seg_i16.reshape(B * N_BLK, 1, PIX_BLK)
    painted = pl.pallas_call(
        _paint_body,
        grid=(B * N_BLK,),
        in_specs=[
            pl.BlockSpec((1, 1, PIX_BLK), lambda i: (i, 0, 0)),
            pl.BlockSpec((1, C, NUM_SEG), lambda i: (i // N_BLK, 0, 0)),
        ],
        out_specs=pl.BlockSpec(
            (1, C, ROWS_BLK, W_img), lambda i: (i // N_BLK, 0, i % N_BLK, 0)),
        out_shape=jax.ShapeDtypeStruct((B, C, H_img, W_img), jnp.float32),
    )(seg_blk, means_t)

    return painted
